# baseline (device time: 873345 ns/iter reference)
import jax
import jax.numpy as jnp
from jax import lax
from jax.experimental import pallas as pl
from jax.experimental.pallas import tpu as pltpu

N_DEV = 8
N_EXP = 64
E_LOC = N_EXP // N_DEV
CAP = 25
CAP_PAD = 32
ROWS = E_LOC * CAP_PAD
D = 512
H = 1024


def _moe_ag_pallas(xg, w):

    def body(xg_ref, w_ref, out_ref, send_sems, recv_sems):
        my = lax.axis_index("i")
        right = (my + 1) % N_DEV

        for j in range(E_LOC):
            y = jnp.dot(
                xg_ref[j * CAP_PAD:(j + 1) * CAP_PAD, :],
                w_ref[j],
                preferred_element_type=jnp.float32,
            )
            out_ref[my, pl.ds(j * CAP_PAD, CAP_PAD), :] = y

        for h in range(N_DEV - 1):
            o = (my - h) % N_DEV
            rdma = pltpu.make_async_remote_copy(
                src_ref=out_ref.at[o],
                dst_ref=out_ref.at[o],
                send_sem=send_sems.at[h],
                recv_sem=recv_sems.at[h],
                device_id=(right,),
                device_id_type=pl.DeviceIdType.MESH,
            )
            rdma.start()
            rdma.wait()

    return pl.pallas_call(
        body,
        out_shape=jax.ShapeDtypeStruct((N_DEV, ROWS, H), jnp.float32),
        in_specs=[
            pl.BlockSpec(memory_space=pltpu.VMEM),
            pl.BlockSpec(memory_space=pltpu.VMEM),
        ],
        out_specs=pl.BlockSpec(memory_space=pltpu.VMEM),
        scratch_shapes=[
            pltpu.SemaphoreType.DMA((N_DEV - 1,)),
            pltpu.SemaphoreType.DMA((N_DEV - 1,)),
        ],
    )(xg, w)


def kernel(x, router_W, route_idx, expert_W):
    del router_W
    n_tok = x.shape[0]
    pos = lax.axis_index("i")

    r = route_idx[:, 0]
    oh = r[:, None] == jnp.arange(N_EXP, dtype=r.dtype)[None, :]
    ranks = jnp.cumsum(oh.astype(jnp.int32), axis=0)
    rank = jnp.sum(jnp.where(oh, ranks, 0), axis=1) - 1
    valid = rank < CAP
    dev = r // E_LOC
    slot = (r % E_LOC) * CAP_PAD + jnp.minimum(rank, CAP_PAD - 1)

    idx_local = jnp.where(valid & (dev == pos), slot, ROWS)
    xg = jnp.zeros((ROWS + 1, D), jnp.float32).at[idx_local].set(x)[:ROWS]

    y_all = _moe_ag_pallas(xg, expert_W)
    y_flat = y_all.reshape(N_DEV * ROWS, H)

    gidx = jnp.where(valid, dev * ROWS + slot, 0)
    return jnp.where(valid[:, None], y_flat[gidx], 0.0)


# device time: 153011 ns/iter; 5.7077x vs baseline; 5.7077x over previous
import jax
import jax.numpy as jnp
from jax import lax
from jax.experimental import pallas as pl
from jax.experimental.pallas import tpu as pltpu

N_DEV = 8
N_EXP = 64
E_LOC = N_EXP // N_DEV
CAP = 25
CAP_PAD = 32
ROWS = E_LOC * CAP_PAD
D = 512
H = 1024
N_TOK = 2048


def _moe_ag_pallas(xg, gidx, w):

    def body(xg_ref, gidx_ref, w_ref, out_ref, comm_ref, send_sems, recv_sems):
        my = lax.axis_index("i")
        right = (my + 1) % N_DEV

        for j in range(E_LOC):
            y = jnp.dot(
                xg_ref[j * CAP_PAD:(j + 1) * CAP_PAD, :],
                w_ref[j],
                preferred_element_type=jnp.float32,
            )
            comm_ref[my, pl.ds(j * CAP_PAD, CAP_PAD), :] = y

        def accum(o, first):
            col = jax.lax.broadcasted_iota(jnp.int32, (N_TOK, ROWS), 1)
            g = (gidx_ref[:, :] == col + o * ROWS).astype(jnp.float32)
            contrib = jnp.dot(
                g, comm_ref[o], preferred_element_type=jnp.float32
            )
            if first:
                out_ref[:, :] = contrib
            else:
                out_ref[:, :] = out_ref[:, :] + contrib

        accum(my, first=True)

        for h in range(N_DEV - 1):
            o_send = (my - h) % N_DEV
            o_recv = (my - h - 1) % N_DEV
            rdma = pltpu.make_async_remote_copy(
                src_ref=comm_ref.at[o_send],
                dst_ref=comm_ref.at[o_send],
                send_sem=send_sems.at[h],
                recv_sem=recv_sems.at[h],
                device_id=(right,),
                device_id_type=pl.DeviceIdType.MESH,
            )
            rdma.start()
            rdma.wait()
            accum(o_recv, first=False)

    return pl.pallas_call(
        body,
        out_shape=jax.ShapeDtypeStruct((N_TOK, H), jnp.float32),
        in_specs=[
            pl.BlockSpec(memory_space=pltpu.VMEM),
            pl.BlockSpec(memory_space=pltpu.VMEM),
            pl.BlockSpec(memory_space=pltpu.VMEM),
        ],
        out_specs=pl.BlockSpec(memory_space=pltpu.VMEM),
        scratch_shapes=[
            pltpu.VMEM((N_DEV, ROWS, H), jnp.float32),
            pltpu.SemaphoreType.DMA((N_DEV - 1,)),
            pltpu.SemaphoreType.DMA((N_DEV - 1,)),
        ],
    )(xg, gidx, w)


def kernel(x, router_W, route_idx, expert_W):
    del router_W
    pos = lax.axis_index("i")

    r = route_idx[:, 0]
    oh = r[:, None] == jnp.arange(N_EXP, dtype=r.dtype)[None, :]
    ranks = jnp.cumsum(oh.astype(jnp.int32), axis=0)
    rank = jnp.sum(jnp.where(oh, ranks, 0), axis=1) - 1
    valid = rank < CAP
    dev = r // E_LOC
    slot = (r % E_LOC) * CAP_PAD + jnp.minimum(rank, CAP_PAD - 1)

    idx_local = jnp.where(valid & (dev == pos), slot, ROWS)
    xg = jnp.zeros((ROWS + 1, D), jnp.float32).at[idx_local].set(x)[:ROWS]

    gidx = jnp.where(valid, dev * ROWS + slot, -1).astype(jnp.int32)

    return _moe_ag_pallas(xg, gidx[:, None], expert_W)


# device time: 104776 ns/iter; 8.3354x vs baseline; 1.4604x over previous
import jax
import jax.numpy as jnp
from jax import lax
from jax.experimental import pallas as pl
from jax.experimental.pallas import tpu as pltpu

N_DEV = 8
N_EXP = 64
E_LOC = N_EXP // N_DEV
CAP = 25
CAP_PAD = 32
ROWS = E_LOC * CAP_PAD
D = 512
H = 1024
N_TOK = 2048


def _moe_ag_pallas(xg, gidx, w):

    def body(xg_ref, gidx_ref, w_ref, out_ref, comm_ref, send_sems, recv_sems):
        my = lax.axis_index("i")
        right = (my + 1) % N_DEV

        for j in range(E_LOC):
            y = jnp.dot(
                xg_ref[j * CAP_PAD:(j + 1) * CAP_PAD, :],
                w_ref[j],
                preferred_element_type=jnp.float32,
            )
            comm_ref[my, pl.ds(j * CAP_PAD, CAP_PAD), :] = y.astype(jnp.bfloat16)

        def accum(o, first):
            col = jax.lax.broadcasted_iota(jnp.int32, (N_TOK, ROWS), 1)
            g = (gidx_ref[:, :] == col + o * ROWS).astype(jnp.bfloat16)
            contrib = jnp.dot(
                g, comm_ref[o], preferred_element_type=jnp.float32
            )
            if first:
                out_ref[:, :] = contrib
            else:
                out_ref[:, :] = out_ref[:, :] + contrib

        for h in range(N_DEV - 1):
            o = (my - h) % N_DEV
            rdma = pltpu.make_async_remote_copy(
                src_ref=comm_ref.at[o],
                dst_ref=comm_ref.at[o],
                send_sem=send_sems.at[h],
                recv_sem=recv_sems.at[h],
                device_id=(right,),
                device_id_type=pl.DeviceIdType.MESH,
            )
            rdma.start()
            accum(o, first=(h == 0))
            rdma.wait()
        accum((my + 1) % N_DEV, first=False)

    return pl.pallas_call(
        body,
        out_shape=jax.ShapeDtypeStruct((N_TOK, H), jnp.float32),
        in_specs=[
            pl.BlockSpec(memory_space=pltpu.VMEM),
            pl.BlockSpec(memory_space=pltpu.VMEM),
            pl.BlockSpec(memory_space=pltpu.VMEM),
        ],
        out_specs=pl.BlockSpec(memory_space=pltpu.VMEM),
        scratch_shapes=[
            pltpu.VMEM((N_DEV, ROWS, H), jnp.bfloat16),
            pltpu.SemaphoreType.DMA((N_DEV - 1,)),
            pltpu.SemaphoreType.DMA((N_DEV - 1,)),
        ],
    )(xg, gidx, w)


def kernel(x, router_W, route_idx, expert_W):
    del router_W
    pos = lax.axis_index("i")

    r = route_idx[:, 0]
    oh = r[:, None] == jnp.arange(N_EXP, dtype=r.dtype)[None, :]
    ranks = jnp.cumsum(oh.astype(jnp.int32), axis=0)
    rank = jnp.sum(jnp.where(oh, ranks, 0), axis=1) - 1
    valid = rank < CAP
    dev = r // E_LOC
    slot = (r % E_LOC) * CAP_PAD + jnp.minimum(rank, CAP_PAD - 1)

    idx_local = jnp.where(valid & (dev == pos), slot, ROWS)
    xg = jnp.zeros((ROWS + 1, D), jnp.float32).at[idx_local].set(x)[:ROWS]

    gidx = jnp.where(valid, dev * ROWS + slot, -1).astype(jnp.int32)

    return _moe_ag_pallas(xg, gidx[:, None], expert_W)


# device time: 75141 ns/iter; 11.6227x vs baseline; 1.3944x over previous
import jax
import jax.numpy as jnp
from jax import lax
from jax.experimental import pallas as pl
from jax.experimental.pallas import tpu as pltpu

N_DEV = 8
N_EXP = 64
E_LOC = N_EXP // N_DEV
CAP = 25
CAP_PAD = 32
ROWS = E_LOC * CAP_PAD
D = 512
H = 1024
N_TOK = 2048


def kernel(x, router_W, route_idx, expert_W):
    del router_W

    def body(x_ref, route_ref, w_ref, out_ref, comm_ref, send_sems, recv_sems):
        my = lax.axis_index("i")
        f32 = jnp.float32
        bf16 = jnp.bfloat16

        r = route_ref[:, :]
        ex = lax.broadcasted_iota(jnp.int32, (N_TOK, N_EXP), 1)
        oh = r == ex
        row = lax.broadcasted_iota(jnp.int32, (N_TOK, N_TOK), 0)
        col = lax.broadcasted_iota(jnp.int32, (N_TOK, N_TOK), 1)
        tril = (col <= row).astype(bf16)
        ranks_incl = jnp.dot(tril, oh.astype(bf16),
                             preferred_element_type=f32)
        rank = jnp.sum(jnp.where(oh, ranks_incl, 0.0), axis=1,
                       keepdims=True).astype(jnp.int32) - 1
        valid = rank < CAP
        dev = r // E_LOC
        slot = (r % E_LOC) * CAP_PAD + jnp.minimum(rank, CAP_PAD - 1)
        gidx = jnp.where(valid, dev * ROWS + slot, -1)

        scol = lax.broadcasted_iota(jnp.int32, (N_TOK, ROWS), 1)
        sT = ((slot == scol) & valid & (dev == my)).astype(bf16)
        xg = lax.dot_general(
            sT, x_ref[:, :].astype(bf16),
            dimension_numbers=(((0,), (0,)), ((), ())),
            preferred_element_type=f32,
        )

        for j in range(E_LOC):
            y = jnp.dot(
                xg[j * CAP_PAD:(j + 1) * CAP_PAD, :].astype(bf16),
                w_ref[j].astype(bf16),
                preferred_element_type=f32,
            )
            comm_ref[my, pl.ds(j * CAP_PAD, CAP_PAD), :] = y.astype(bf16)

        sends = []
        for k in range(1, N_DEV):
            tgt = (my + k) % N_DEV
            s = pltpu.make_async_remote_copy(
                src_ref=comm_ref.at[my],
                dst_ref=comm_ref.at[my],
                send_sem=send_sems.at[k - 1],
                recv_sem=recv_sems.at[my],
                device_id=(tgt,),
                device_id_type=pl.DeviceIdType.MESH,
            )
            s.start()
            sends.append(s)

        def accum(o, first):
            gcol = lax.broadcasted_iota(jnp.int32, (N_TOK, ROWS), 1)
            g = (gidx == gcol + o * ROWS).astype(bf16)
            contrib = jnp.dot(g, comm_ref[o], preferred_element_type=f32)
            if first:
                out_ref[:, :] = contrib
            else:
                out_ref[:, :] = out_ref[:, :] + contrib

        accum(my, first=True)

        for k in range(1, N_DEV):
            o = (my + k) % N_DEV
            recv = pltpu.make_async_remote_copy(
                src_ref=comm_ref.at[o],
                dst_ref=comm_ref.at[o],
                send_sem=send_sems.at[k - 1],
                recv_sem=recv_sems.at[o],
                device_id=(my,),
                device_id_type=pl.DeviceIdType.MESH,
            )
            recv.wait_recv()
            accum(o, first=False)

        for s in sends:
            s.wait_send()

    return pl.pallas_call(
        body,
        out_shape=jax.ShapeDtypeStruct((N_TOK, H), jnp.float32),
        in_specs=[
            pl.BlockSpec(memory_space=pltpu.VMEM),
            pl.BlockSpec(memory_space=pltpu.VMEM),
            pl.BlockSpec(memory_space=pltpu.VMEM),
        ],
        out_specs=pl.BlockSpec(memory_space=pltpu.VMEM),
        scratch_shapes=[
            pltpu.VMEM((N_DEV, ROWS, H), jnp.bfloat16),
            pltpu.SemaphoreType.DMA((N_DEV - 1,)),
            pltpu.SemaphoreType.DMA((N_DEV,)),
        ],
    )(x, route_idx, expert_W)


# device time: 72018 ns/iter; 12.1268x vs baseline; 1.0434x over previous
import jax
import jax.numpy as jnp
from jax import lax
from jax.experimental import pallas as pl
from jax.experimental.pallas import tpu as pltpu

N_DEV = 8
N_EXP = 64
E_LOC = N_EXP // N_DEV
CAP = 25
CAP_PAD = 32
ROWS = E_LOC * CAP_PAD
D = 512
H = 1024
N_TOK = 2048
SLOTS = N_DEV * ROWS


def kernel(x, router_W, route_idx, expert_W):
    del router_W

    def body(x_ref, route_ref, w_ref, out_ref, comm_ref, g_ref,
             send_sems, recv_sems):
        my = lax.axis_index("i")
        f32 = jnp.float32
        bf16 = jnp.bfloat16

        r = route_ref[:, :]
        ex = lax.broadcasted_iota(jnp.int32, (N_TOK, N_EXP), 1)
        oh = r == ex
        row = lax.broadcasted_iota(jnp.int32, (N_TOK, N_TOK), 0)
        col = lax.broadcasted_iota(jnp.int32, (N_TOK, N_TOK), 1)
        tril = (col <= row).astype(bf16)
        ranks_incl = jnp.dot(tril, oh.astype(bf16),
                             preferred_element_type=f32)
        rank = jnp.sum(jnp.where(oh, ranks_incl, 0.0), axis=1,
                       keepdims=True).astype(jnp.int32) - 1
        valid = rank < CAP
        dev = r // E_LOC
        slot = (r % E_LOC) * CAP_PAD + jnp.minimum(rank, CAP_PAD - 1)
        gidx = jnp.where(valid, dev * ROWS + slot, -1)

        scol = lax.broadcasted_iota(jnp.int32, (N_TOK, ROWS), 1)
        sT = (gidx == scol + my * ROWS).astype(bf16)
        xg = lax.dot_general(
            sT, x_ref[:, :].astype(bf16),
            dimension_numbers=(((0,), (0,)), ((), ())),
            preferred_element_type=f32,
        )

        for j in range(E_LOC):
            y = jnp.dot(
                xg[j * CAP_PAD:(j + 1) * CAP_PAD, :].astype(bf16),
                w_ref[j].astype(bf16),
                preferred_element_type=f32,
            )
            comm_ref[pl.ds(my * ROWS + j * CAP_PAD, CAP_PAD), :] = (
                y.astype(bf16)
            )

        sends = []
        for k in range(1, N_DEV):
            tgt = (my + k) % N_DEV
            s = pltpu.make_async_remote_copy(
                src_ref=comm_ref.at[pl.ds(my * ROWS, ROWS), :],
                dst_ref=comm_ref.at[pl.ds(my * ROWS, ROWS), :],
                send_sem=send_sems.at[k - 1],
                recv_sem=recv_sems.at[my],
                device_id=(tgt,),
                device_id_type=pl.DeviceIdType.MESH,
            )
            s.start()
            sends.append(s)

        gcol = lax.broadcasted_iota(jnp.int32, (N_TOK, SLOTS), 1)
        g_ref[:, :] = (gidx == gcol).astype(bf16)

        for k in range(1, N_DEV):
            o = (my + k) % N_DEV
            recv = pltpu.make_async_remote_copy(
                src_ref=comm_ref.at[pl.ds(o * ROWS, ROWS), :],
                dst_ref=comm_ref.at[pl.ds(o * ROWS, ROWS), :],
                send_sem=send_sems.at[k - 1],
                recv_sem=recv_sems.at[o],
                device_id=(my,),
                device_id_type=pl.DeviceIdType.MESH,
            )
            recv.wait_recv()

        out_ref[:, :] = jnp.dot(g_ref[:, :], comm_ref[:, :],
                                preferred_element_type=f32)

        for s in sends:
            s.wait_send()

    return pl.pallas_call(
        body,
        out_shape=jax.ShapeDtypeStruct((N_TOK, H), jnp.float32),
        in_specs=[
            pl.BlockSpec(memory_space=pltpu.VMEM),
            pl.BlockSpec(memory_space=pltpu.VMEM),
            pl.BlockSpec(memory_space=pltpu.VMEM),
        ],
        out_specs=pl.BlockSpec(memory_space=pltpu.VMEM),
        scratch_shapes=[
            pltpu.VMEM((SLOTS, H), jnp.bfloat16),
            pltpu.VMEM((N_TOK, SLOTS), jnp.bfloat16),
            pltpu.SemaphoreType.DMA((N_DEV - 1,)),
            pltpu.SemaphoreType.DMA((N_DEV,)),
        ],
    )(x, route_idx, expert_W)


# device time: 66151 ns/iter; 13.2023x vs baseline; 1.0887x over previous
import jax
import jax.numpy as jnp
from jax import lax
from jax.experimental import pallas as pl
from jax.experimental.pallas import tpu as pltpu

N_DEV = 8
N_EXP = 64
E_LOC = N_EXP // N_DEV
CAP = 25
CAP_PAD = 32
ROWS = E_LOC * CAP_PAD
D = 512
H = 1024
N_TOK = 2048
SLOTS = N_DEV * ROWS


def kernel(x, router_W, route_idx, expert_W):
    del router_W

    def body(x_ref, route_ref, w_ref, out_ref, comm_ref, g_ref,
             send_sems, recv_sems):
        my = lax.axis_index("i")
        f32 = jnp.float32
        bf16 = jnp.bfloat16

        with jax.named_scope("rank"):
            r = route_ref[:, :]
            ex = lax.broadcasted_iota(jnp.int32, (N_TOK, N_EXP), 1)
            oh = r == ex
            ranks_incl = oh.astype(jnp.int32)
            s = 1
            while s < N_TOK:
                shifted = jnp.concatenate(
                    [jnp.zeros((s, N_EXP), jnp.int32), ranks_incl[:-s, :]],
                    axis=0,
                )
                ranks_incl = ranks_incl + shifted
                s *= 2
            rank = jnp.sum(jnp.where(oh, ranks_incl, 0), axis=1,
                           keepdims=True) - 1
            valid = rank < CAP
            dev = r // E_LOC
            slot = (r % E_LOC) * CAP_PAD + jnp.minimum(rank, CAP_PAD - 1)
            gidx = jnp.where(valid, dev * ROWS + slot, -1)

        with jax.named_scope("compact"):
            scol = lax.broadcasted_iota(jnp.int32, (N_TOK, ROWS), 1)
            sT = (gidx == scol + my * ROWS).astype(bf16)
            xg = lax.dot_general(
                sT, x_ref[:, :].astype(bf16),
                dimension_numbers=(((0,), (0,)), ((), ())),
                preferred_element_type=f32,
            )

        with jax.named_scope("expert_mm"):
            for j in range(E_LOC):
                y = jnp.dot(
                    xg[j * CAP_PAD:(j + 1) * CAP_PAD, :].astype(bf16),
                    w_ref[j].astype(bf16),
                    preferred_element_type=f32,
                )
                comm_ref[pl.ds(my * ROWS + j * CAP_PAD, CAP_PAD), :] = (
                    y.astype(bf16)
                )

        with jax.named_scope("send"):
            sends = []
            for k in range(1, N_DEV):
                tgt = (my + k) % N_DEV
                s = pltpu.make_async_remote_copy(
                    src_ref=comm_ref.at[pl.ds(my * ROWS, ROWS), :],
                    dst_ref=comm_ref.at[pl.ds(my * ROWS, ROWS), :],
                    send_sem=send_sems.at[k - 1],
                    recv_sem=recv_sems.at[my],
                    device_id=(tgt,),
                    device_id_type=pl.DeviceIdType.MESH,
                )
                s.start()
                sends.append(s)

        with jax.named_scope("gbuild"):
            gcol = lax.broadcasted_iota(jnp.int32, (N_TOK, SLOTS), 1)
            g_ref[:, :] = (gidx == gcol).astype(bf16)

        with jax.named_scope("waitrecv"):
            for k in range(1, N_DEV):
                o = (my + k) % N_DEV
                recv = pltpu.make_async_remote_copy(
                    src_ref=comm_ref.at[pl.ds(o * ROWS, ROWS), :],
                    dst_ref=comm_ref.at[pl.ds(o * ROWS, ROWS), :],
                    send_sem=send_sems.at[k - 1],
                    recv_sem=recv_sems.at[o],
                    device_id=(my,),
                    device_id_type=pl.DeviceIdType.MESH,
                )
                recv.wait_recv()

        with jax.named_scope("final_mm"):
            out_ref[:, :] = jnp.dot(g_ref[:, :], comm_ref[:, :],
                                    preferred_element_type=f32)

        with jax.named_scope("waitsend"):
            for s in sends:
                s.wait_send()

    return pl.pallas_call(
        body,
        out_shape=jax.ShapeDtypeStruct((N_TOK, H), jnp.float32),
        in_specs=[
            pl.BlockSpec(memory_space=pltpu.VMEM),
            pl.BlockSpec(memory_space=pltpu.VMEM),
            pl.BlockSpec(memory_space=pltpu.VMEM),
        ],
        out_specs=pl.BlockSpec(memory_space=pltpu.VMEM),
        scratch_shapes=[
            pltpu.VMEM((SLOTS, H), jnp.bfloat16),
            pltpu.VMEM((N_TOK, SLOTS), jnp.bfloat16),
            pltpu.SemaphoreType.DMA((N_DEV - 1,)),
            pltpu.SemaphoreType.DMA((N_DEV,)),
        ],
    )(x, route_idx, expert_W)
